# baseline (speedup 1.0000x reference)
import jax
import jax.numpy as jnp
from jax.experimental import pallas as pl
from jax.experimental.pallas import tpu as pltpu
from functools import partial

B, L = 4096, 200
VOCAB_SIZE, CONCEPT_DIM = 32, 8
N_EMB, D_EMB = 64, 16

BLOCK_B = 128


def _kern(ids_ref, embed_ref, wh_ref, bh_ref, wc_ref, bc_ref,
          logits_ref, conc_ref):
    ids = ids_ref[...]
    ids3 = ids.reshape(BLOCK_B, 1, L)
    e_iota = jax.lax.broadcasted_iota(jnp.int32, (1, N_EMB, 1), 1)
    onehot = (ids3 == e_iota).astype(jnp.float32)
    oh2 = onehot.reshape(BLOCK_B * N_EMB, L)
    ones_v = jnp.full((L, 8), 1.0, dtype=jnp.float32)
    r = jnp.dot(oh2, ones_v, preferred_element_type=jnp.float32)
    counts = r[:, :1].reshape(BLOCK_B, N_EMB)
    x = jnp.dot(counts, embed_ref[...], preferred_element_type=jnp.float32)
    x = x * (1.0 / L)
    logits = jnp.dot(x, wh_ref[...], preferred_element_type=jnp.float32)
    logits = logits + bh_ref[...]
    conc = jnp.dot(x, wc_ref[...], preferred_element_type=jnp.float32)
    conc = conc + bc_ref[...]
    logits_ref[...] = jnp.broadcast_to(
        logits[:, None, :], (BLOCK_B, L, VOCAB_SIZE))
    conc_ref[...] = conc


@jax.jit
def kernel(input_ids, embed, W_head, b_head, W_concept, b_concept):
    grid = (B // BLOCK_B,)
    logits2d, concepts = pl.pallas_call(
        _kern,
        grid=grid,
        in_specs=[
            pl.BlockSpec((BLOCK_B, L), lambda i: (i, 0)),
            pl.BlockSpec((N_EMB, D_EMB), lambda i: (0, 0)),
            pl.BlockSpec((D_EMB, VOCAB_SIZE), lambda i: (0, 0)),
            pl.BlockSpec((1, VOCAB_SIZE), lambda i: (0, 0)),
            pl.BlockSpec((D_EMB, CONCEPT_DIM), lambda i: (0, 0)),
            pl.BlockSpec((1, CONCEPT_DIM), lambda i: (0, 0)),
        ],
        out_specs=[
            pl.BlockSpec((BLOCK_B, L, VOCAB_SIZE), lambda i: (i, 0, 0)),
            pl.BlockSpec((BLOCK_B, CONCEPT_DIM), lambda i: (i, 0)),
        ],
        out_shape=[
            jax.ShapeDtypeStruct((B, L, VOCAB_SIZE), jnp.float32),
            jax.ShapeDtypeStruct((B, CONCEPT_DIM), jnp.float32),
        ],
    )(input_ids, embed, W_head, b_head.reshape(1, VOCAB_SIZE),
      W_concept, b_concept.reshape(1, CONCEPT_DIM))
    logits = logits2d
    vertex_preds = jnp.zeros((B, L), dtype=jnp.int32)
    return (logits, concepts, vertex_preds)


# speedup vs baseline: 4.3276x; 4.3276x over previous
import jax
import jax.numpy as jnp
from jax.experimental import pallas as pl
from jax.experimental.pallas import tpu as pltpu
from functools import partial

B, L = 4096, 200
VOCAB_SIZE, CONCEPT_DIM = 32, 8
N_EMB, D_EMB = 64, 16

BLOCK_B = 256


def _kern(ids_ref, embed_ref, wh_ref, bh_ref, wc_ref, bc_ref,
          logits_ref, conc_ref):
    ids = ids_ref[...]
    ids3 = ids.reshape(BLOCK_B, 1, L)
    e_iota = jax.lax.broadcasted_iota(jnp.int32, (1, N_EMB, 1), 1)
    onehot = (ids3 == e_iota).astype(jnp.float32)
    oh2 = onehot.reshape(BLOCK_B * N_EMB, L)
    ones_v = jnp.full((L, 8), 1.0, dtype=jnp.float32)
    r = jnp.dot(oh2, ones_v, preferred_element_type=jnp.float32)
    counts = r[:, :1].reshape(BLOCK_B, N_EMB)
    x = jnp.dot(counts, embed_ref[...], preferred_element_type=jnp.float32)
    x = x * (1.0 / L)
    logits = jnp.dot(x, wh_ref[...], preferred_element_type=jnp.float32)
    logits = logits + bh_ref[...]
    conc = jnp.dot(x, wc_ref[...], preferred_element_type=jnp.float32)
    conc = conc + bc_ref[...]
    logits_ref[...] = logits
    conc_ref[...] = conc


@jax.jit
def kernel(input_ids, embed, W_head, b_head, W_concept, b_concept):
    grid = (B // BLOCK_B,)
    logits2d, concepts = pl.pallas_call(
        _kern,
        grid=grid,
        in_specs=[
            pl.BlockSpec((BLOCK_B, L), lambda i: (i, 0)),
            pl.BlockSpec((N_EMB, D_EMB), lambda i: (0, 0)),
            pl.BlockSpec((D_EMB, VOCAB_SIZE), lambda i: (0, 0)),
            pl.BlockSpec((1, VOCAB_SIZE), lambda i: (0, 0)),
            pl.BlockSpec((D_EMB, CONCEPT_DIM), lambda i: (0, 0)),
            pl.BlockSpec((1, CONCEPT_DIM), lambda i: (0, 0)),
        ],
        out_specs=[
            pl.BlockSpec((BLOCK_B, VOCAB_SIZE), lambda i: (i, 0)),
            pl.BlockSpec((BLOCK_B, CONCEPT_DIM), lambda i: (i, 0)),
        ],
        out_shape=[
            jax.ShapeDtypeStruct((B, VOCAB_SIZE), jnp.float32),
            jax.ShapeDtypeStruct((B, CONCEPT_DIM), jnp.float32),
        ],
    )(input_ids, embed, W_head, b_head.reshape(1, VOCAB_SIZE),
      W_concept, b_concept.reshape(1, CONCEPT_DIM))
    logits = jnp.broadcast_to(logits2d[:, None, :], (B, L, VOCAB_SIZE))
    vertex_preds = jnp.zeros((B, L), dtype=jnp.int32)
    return (logits, concepts, vertex_preds)
